# baseline (speedup 1.0000x reference)
import functools

import jax
import jax.numpy as jnp
from jax.experimental import pallas as pl
from jax.experimental.pallas import tpu as pltpu

N_CENTER = 512
N_NEAR = 32
N_STEPK = 10


def _index_points(points, idx):
    B = points.shape[0]
    bidx = jnp.arange(B).reshape((B,) + (1,) * (idx.ndim - 1))
    bidx = jnp.broadcast_to(bidx, idx.shape)
    return points[bidx, idx]


def _farthest_point_sample(xyz, n_samples, key):
    B, N, _ = xyz.shape
    distance = jnp.full((B, N), 1e10, dtype=xyz.dtype)
    farthest = jax.random.randint(key, (B,), 0, N)
    batch_indices = jnp.arange(B)

    def body(carry, _):
        distance, farthest = carry
        centroid = xyz[batch_indices, farthest, :][:, None, :]
        dist = jnp.sum((xyz - centroid) ** 2, -1)
        distance = jnp.where(dist < distance, dist, distance)
        new_farthest = jnp.argmax(distance, -1)
        return (distance, new_farthest), farthest

    (_, _), centroids = jax.lax.scan(body, (distance, farthest), None,
                                     length=n_samples)
    return jnp.transpose(centroids)


def _get_neighbor_index(vertices, neighbor_num):
    inner = jnp.einsum('bnd,bmd->bnm', vertices, vertices)
    quadratic = jnp.sum(vertices ** 2, axis=2)
    distance = inner * -2 + quadratic[:, None, :] + quadratic[:, :, None]
    neighbor_index = jax.lax.top_k(-distance, neighbor_num + 1)[1][:, :, 1:]
    return neighbor_index, distance


def _surface_knn(points_all, k_near, n_stepk):
    ind_neighbor_all, all_dist = _get_neighbor_index(points_all, n_stepk)
    neighbor_index_max = jnp.argmax(all_dist, axis=-1, keepdims=True)
    B, N, _ = ind_neighbor_all.shape
    K = k_near
    pad_val = jnp.int32(N)
    parent_ids = jnp.repeat(jnp.arange(K), n_stepk)
    init_neigh = jnp.concatenate(
        [ind_neighbor_all,
         jnp.full((B, N, K - n_stepk), pad_val, dtype=ind_neighbor_all.dtype)],
        axis=-1)

    def body(state):
        neigh, w, num_ita, done = state
        children = _index_points(ind_neighbor_all, neigh).reshape(B, N, K * n_stepk)
        children = jnp.where(parent_ids[None, None, :] < w, children, pad_val)
        new_neighinds = jnp.sort(children, axis=-1)
        duplicates = jnp.zeros(new_neighinds.shape, dtype=bool)
        duplicates = duplicates.at[:, :, 1:].set(
            new_neighinds[:, :, 1:] == new_neighinds[:, :, :-1])
        nimax = jnp.broadcast_to(neighbor_index_max, new_neighinds.shape)
        new_neighinds = jnp.where(duplicates, nimax, new_neighinds)
        new_neighinds = jnp.where(new_neighinds == pad_val, nimax, new_neighinds)
        dist_neighinds = jnp.take_along_axis(all_dist, new_neighinds, axis=-1)
        sort_dist = jnp.sort(dist_neighinds, axis=-1)
        L = jnp.int32(n_stepk) * w
        last_idx = jnp.full((B, N, 1), L - 1, dtype=jnp.int32)
        row_max = jnp.take_along_axis(sort_dist, last_idx, axis=-1)
        sort_dist_maxind = jnp.argmax(sort_dist == row_max, axis=-1)
        valid_raw = jnp.min(sort_dist_maxind) + 1
        is_end = valid_raw >= k_near + 1
        valid_nnear = jnp.minimum(valid_raw, k_near + 1)
        sub_neighbor_index = jax.lax.top_k(-dist_neighinds, k_near + 1)[1]
        new_neighinds = jnp.take_along_axis(new_neighinds, sub_neighbor_index,
                                            axis=-1)
        new_neighinds = jnp.where(
            jnp.arange(k_near + 1)[None, None, :] < valid_nnear,
            new_neighinds, pad_val)
        new_neighinds = new_neighinds[:, :, 1:]
        return new_neighinds, valid_nnear - 1, num_ita + 1, is_end

    def cond(state):
        _, _, num_ita, done = state
        return jnp.logical_not(done) & (num_ita < 21)

    state = (init_neigh, jnp.int32(n_stepk), jnp.int32(0), jnp.bool_(False))
    neigh, _, _, _ = jax.lax.while_loop(cond, body, state)
    return neigh


def _layer_kernel(x_ref, w_ref, b_ref, scale_ref, shift_ref, z_ref,
                  sum_ref, sq_ref, *, first):
    j = pl.program_id(0)
    x = x_ref[...]
    if not first:
        x = jnp.maximum(x * scale_ref[...][:, :1] + shift_ref[...][:, :1], 0.0)
    z = jnp.dot(w_ref[...], x, preferred_element_type=jnp.float32)
    z = z + b_ref[...][:, :1]
    z_ref[...] = z

    @pl.when(j == 0)
    def _init():
        sum_ref[...] = jnp.zeros_like(sum_ref)
        sq_ref[...] = jnp.zeros_like(sq_ref)

    sum_ref[...] += jnp.sum(z, axis=1, keepdims=True)
    sq_ref[...] += jnp.sum(z * z, axis=1, keepdims=True)


def _mlp_layer(x, w, b, scale, shift, *, first, chunk=8192):
    cin, cols = x.shape
    cout = w.shape[0]
    grid = cols // chunk
    z, s, sq = pl.pallas_call(
        functools.partial(_layer_kernel, first=first),
        grid=(grid,),
        in_specs=[
            pl.BlockSpec((cin, chunk), lambda j: (0, j)),
            pl.BlockSpec((cout, cin), lambda j: (0, 0)),
            pl.BlockSpec((cout, 1), lambda j: (0, 0)),
            pl.BlockSpec((cin, 1), lambda j: (0, 0)),
            pl.BlockSpec((cin, 1), lambda j: (0, 0)),
        ],
        out_specs=[
            pl.BlockSpec((cout, chunk), lambda j: (0, j)),
            pl.BlockSpec((cout, 1), lambda j: (0, 0)),
            pl.BlockSpec((cout, 1), lambda j: (0, 0)),
        ],
        out_shape=[
            jax.ShapeDtypeStruct((cout, cols), jnp.float32),
            jax.ShapeDtypeStruct((cout, 1), jnp.float32),
            jax.ShapeDtypeStruct((cout, 1), jnp.float32),
        ],
    )(x, w, b.reshape(-1, 1), scale.reshape(-1, 1), shift.reshape(-1, 1))
    return z, s[:, 0], sq[:, 0]


def _bn_coeffs(s, sq, n, g, be):
    mean = s / n
    var = sq / n - mean * mean
    inv = g / jnp.sqrt(var + 1e-5)
    return inv, be - mean * inv


def _final_kernel(z_ref, scale_ref, shift_ref, o_ref):
    a = jnp.maximum(z_ref[...] * scale_ref[...][:, :1] + shift_ref[...][:, :1],
                    0.0)
    c, n = a.shape
    a = a.reshape(c, N_NEAR, n // N_NEAR)
    o_ref[...] = jnp.max(a, axis=1)


def _mlp_head(new_fea_t, params):
    (W1, b1, g1, be1, W2, b2, g2, be2, W3, b3, g3, be3) = params
    cols = new_fea_t.shape[1]
    n = float(cols)
    zero67 = jnp.zeros((67,), jnp.float32)
    z1, s1, q1 = _mlp_layer(new_fea_t, W1, b1, zero67, zero67, first=True)
    sc1, sh1 = _bn_coeffs(s1, q1, n, g1, be1)
    z2, s2, q2 = _mlp_layer(z1, W2, b2, sc1, sh1, first=False)
    sc2, sh2 = _bn_coeffs(s2, q2, n, g2, be2)
    z3, s3, q3 = _mlp_layer(z2, W3, b3, sc2, sh2, first=False)
    sc3, sh3 = _bn_coeffs(s3, q3, n, g3, be3)
    B = cols // (N_NEAR * N_CENTER)
    chunk = N_NEAR * N_CENTER
    out = pl.pallas_call(
        _final_kernel,
        grid=(B,),
        in_specs=[
            pl.BlockSpec((128, chunk), lambda j: (0, j)),
            pl.BlockSpec((128, 1), lambda j: (0, 0)),
            pl.BlockSpec((128, 1), lambda j: (0, 0)),
        ],
        out_specs=pl.BlockSpec((128, N_CENTER), lambda j: (0, j)),
        out_shape=jax.ShapeDtypeStruct((128, B * N_CENTER), jnp.float32),
    )(z3, sc3.reshape(-1, 1), sh3.reshape(-1, 1))
    return out.reshape(128, B, N_CENTER).transpose(1, 0, 2)


def kernel(xyz, points, W1, b1, g1, be1, W2, b2, g2, be2, W3, b3, g3, be3):
    B = xyz.shape[0]
    xyz_t = jnp.transpose(xyz, (0, 2, 1))
    pts_t = jnp.transpose(points, (0, 2, 1))

    idx_surfknn_all = _surface_knn(xyz_t, N_NEAR, N_STEPK)
    fps_idx = _farthest_point_sample(xyz_t, N_CENTER, jax.random.key(1))
    new_xyz = _index_points(xyz_t, fps_idx)
    idx = _index_points(idx_surfknn_all, fps_idx)
    grouped_xyz = _index_points(xyz_t, idx)
    grouped_xyz_norm = grouped_xyz - new_xyz[:, :, None, :]
    grouped_fea = _index_points(pts_t, idx)
    new_fea = jnp.concatenate([grouped_xyz_norm, grouped_fea], axis=-1)

    x = new_fea.transpose(3, 0, 2, 1).reshape(67, B * N_NEAR * N_CENTER)
    params = (W1, b1, g1, be1, W2, b2, g2, be2, W3, b3, g3, be3)
    new_points = _mlp_head(x, params)
    return jnp.transpose(new_xyz, (0, 2, 1)), new_points
